# baseline (speedup 1.0000x reference)
import functools

import jax
import jax.numpy as jnp
from jax import lax
from jax.experimental import pallas as pl
from jax.experimental.pallas import tpu as pltpu
from jax.experimental.pallas import tpu_sc as plsc

NC = 2
NS = 16
LANES = 16
NBUF = 5


def _sc_segment_kernel(n_pad: int, e: int, dh: int, batch: int):
    chunk = e // NS
    nb = chunk // batch
    rows_per_tile = n_pad // NS
    stage = 32
    n_stage = rows_per_tile // stage
    groups = [(g * LANES, 0, LANES) for g in range(batch // LANES)]
    if batch % LANES:
        groups.append((batch - LANES, LANES - batch % LANES, batch % LANES))

    mesh = plsc.VectorSubcoreMesh(
        core_axis_name="c", subcore_axis_name="s",
        num_cores=NC, num_subcores=NS)

    @functools.partial(
        pl.kernel,
        out_type=(
            jax.ShapeDtypeStruct((n_pad, NC, dh), jnp.float32),
            jax.ShapeDtypeStruct((n_pad, LANES), jnp.float32),
        ),
        mesh=mesh,
        compiler_params=pltpu.CompilerParams(
            use_tc_tiling_on_sc=False, needs_layout_passes=False),
        scratch_types=[
            pltpu.VMEM((nb, batch), jnp.int32),
            pltpu.VMEM((nb, batch), jnp.int32),
            pltpu.VMEM((nb, batch), jnp.float32),
            pltpu.VMEM((NBUF, batch, dh), jnp.float32),
            pltpu.VMEM((NBUF, batch, LANES), jnp.float32),
            pltpu.VMEM((stage, dh), jnp.float32),
            pltpu.VMEM((stage, LANES), jnp.float32),
            pltpu.VMEM_SHARED((n_pad, dh), jnp.float32),
            pltpu.VMEM_SHARED((n_pad, LANES), jnp.float32),
            [pltpu.SemaphoreType.DMA] * NBUF,
            [pltpu.SemaphoreType.DMA] * NBUF,
        ],
    )
    def sc_kernel(x_hbm, eb_hbm, ea_hbm, w_hbm, g_out, s_out,
                  idx_b, idx_a, w_v, rows_v, srow_v, stage_g, stage_s,
                  g_acc, s_acc, gsem, ssem):
        cid = lax.axis_index("c")
        sid = lax.axis_index("s")
        on_c0 = cid == 0
        zero16 = jnp.zeros((LANES,), jnp.float32)

        def zg(i, _):
            for j in range(dh // LANES):
                stage_g[i, pl.ds(j * LANES, LANES)] = zero16
            stage_s[i, :] = zero16
            return 0
        lax.fori_loop(0, stage, zg, 0)
        for t in range(n_stage):
            row0 = sid * rows_per_tile + t * stage
            pltpu.sync_copy(stage_g, g_acc.at[pl.ds(row0, stage)])
            pltpu.sync_copy(stage_s, s_acc.at[pl.ds(row0, stage)])
        plsc.subcore_barrier()

        pltpu.sync_copy(eb_hbm.at[cid].at[sid], idx_b)
        pltpu.sync_copy(ea_hbm.at[sid], idx_a)
        pltpu.sync_copy(w_hbm.at[sid], w_v)

        def issue_gather(k, p):
            pltpu.async_copy(x_hbm.at[idx_b.at[k]], rows_v.at[p], gsem[p])

        def wait_gather(k, p):
            pltpu.make_async_copy(x_hbm.at[idx_b.at[k]],
                                  rows_v.at[p], gsem[p]).wait()

        def issue_scatter(k, p):
            pltpu.async_copy(rows_v.at[p], g_acc.at[idx_a.at[k]],
                             ssem[p], add=True)

            @pl.when(on_c0)
            def _():
                pltpu.async_copy(srow_v.at[p], s_acc.at[idx_a.at[k]],
                                 ssem[p], add=True)

        def wait_scatter(k, p):
            pltpu.make_async_copy(rows_v.at[p], g_acc.at[idx_a.at[k]],
                                  ssem[p]).wait()

            @pl.when(on_c0)
            def _():
                pltpu.make_async_copy(srow_v.at[p], s_acc.at[idx_a.at[k]],
                                      ssem[p]).wait()

        iota16 = lax.iota(jnp.int32, LANES)

        def scale(k, p):
            for (off, lo, cnt) in groups:
                wvec = w_v[k, pl.ds(off, LANES)]
                wvs = [jnp.full((LANES,), wvec[lo + t], jnp.float32)
                       for t in range(cnt)]
                for t in range(cnt):
                    srow_v[p, off + lo + t, :] = wvs[t]
                for t in range(cnt):
                    for j in range(dh // LANES):
                        sl = pl.ds(j * LANES, LANES)
                        i = off + lo + t
                        rows_v[p, i, sl] = rows_v[p, i, sl] * wvs[t]

        issue_gather(0, 0)
        issue_gather(1, 1)
        n_it = nb // NBUF

        def body(m, _):
            for p in range(NBUF):
                k = m * NBUF + p
                q = (p + 2) % NBUF

                if p >= NBUF - 2:
                    wait_scatter(k - (NBUF - 2), q)
                else:
                    @pl.when(m >= 1)
                    def _():
                        wait_scatter(k - (NBUF - 2), q)

                if p < NBUF - 2:
                    issue_gather(k + 2, q)
                else:
                    @pl.when(m < n_it - 1)
                    def _():
                        issue_gather(k + 2, q)

                wait_gather(k, p)
                scale(k, p)
                issue_scatter(k, p)
            return 0
        lax.fori_loop(0, n_it, body, 0)
        for k in range(nb - (NBUF - 2), nb):
            wait_scatter(k, k % NBUF)
        plsc.subcore_barrier()

        for t in range(n_stage):
            row0 = sid * rows_per_tile + t * stage
            pltpu.sync_copy(g_acc.at[pl.ds(row0, stage)], stage_g)
            pltpu.sync_copy(stage_g, g_out.at[pl.ds(row0, stage), cid])

            @pl.when(on_c0)
            def _():
                pltpu.sync_copy(s_acc.at[pl.ds(row0, stage)], stage_s)
                pltpu.sync_copy(stage_s, s_out.at[pl.ds(row0, stage)])

    return sc_kernel


def _tc_body(x_ref, g_ref, s_ref, w_ref, b_ref, o_ref, *, in_c):
    w1 = w_ref[0:in_c, :]
    w2 = w_ref[in_c:2 * in_c, :]
    x = x_ref[...]
    g = g_ref[...]
    sv = s_ref[:, 0:1]
    h = jnp.dot(x, w1 - w2, preferred_element_type=jnp.float32) + b_ref[...]
    y = sv * h + jnp.dot(g, w2, preferred_element_type=jnp.float32)
    o_ref[...] = jnp.where(y >= 0, y, 0.3 * y)


def kernel(input_features, edges_a, edges_b, adj_weights, W, b):
    n, in_c = input_features.shape
    e = edges_a.shape[0]
    out_c = W.shape[1]
    dh = in_c // NC
    batch = 32
    nb = e // NS // batch

    x_view = input_features.reshape(n * NC, dh)
    eb2 = edges_b * 2
    eb_r = jnp.stack([eb2, eb2 + 1]).reshape(NC, NS, nb, batch)
    ea_r = edges_a.reshape(NS, nb, batch)
    w_r = adj_weights.reshape(NS, nb, batch)

    n_pad = ((n + NS * 128 - 1) // (NS * 128)) * (NS * 128)
    g_parts, s_part = _sc_segment_kernel(n_pad, e, dh, batch)(
        x_view, eb_r, ea_r, w_r)
    g_full = g_parts.reshape(n_pad, in_c)

    bn = 1000
    out = pl.pallas_call(
        functools.partial(_tc_body, in_c=in_c),
        grid=(n // bn,),
        in_specs=[
            pl.BlockSpec((bn, in_c), lambda i: (i, 0)),
            pl.BlockSpec((bn, in_c), lambda i: (i, 0)),
            pl.BlockSpec((bn, LANES), lambda i: (i, 0)),
            pl.BlockSpec((2 * in_c, out_c), lambda i: (0, 0)),
            pl.BlockSpec((1, out_c), lambda i: (0, 0)),
        ],
        out_specs=pl.BlockSpec((bn, out_c), lambda i: (i, 0)),
        out_shape=jax.ShapeDtypeStruct((n, out_c), jnp.float32),
    )(input_features, g_full, s_part, W, b.reshape(1, out_c))
    return out


# speedup vs baseline: 1.0989x; 1.0989x over previous
import functools

import jax
import jax.numpy as jnp
from jax import lax
from jax.experimental import pallas as pl
from jax.experimental.pallas import tpu as pltpu
from jax.experimental.pallas import tpu_sc as plsc

NC = 2
NS = 16
LANES = 16
NBUF = 5


def _sc_segment_kernel(n_pad: int, e: int, dh: int, batch: int):
    chunk = e // NS
    nb = chunk // batch
    rows_per_tile = n_pad // NS
    stage = 64
    n_stage = rows_per_tile // stage
    groups = [(g * LANES, 0, LANES) for g in range(batch // LANES)]
    if batch % LANES:
        groups.append((batch - LANES, LANES - batch % LANES, batch % LANES))

    mesh = plsc.VectorSubcoreMesh(
        core_axis_name="c", subcore_axis_name="s",
        num_cores=NC, num_subcores=NS)

    @functools.partial(
        pl.kernel,
        out_type=(
            jax.ShapeDtypeStruct((n_pad, NC, dh), jnp.float32),
            jax.ShapeDtypeStruct((n_pad, LANES), jnp.float32),
        ),
        mesh=mesh,
        compiler_params=pltpu.CompilerParams(
            use_tc_tiling_on_sc=False, needs_layout_passes=False),
        scratch_types=[
            pltpu.VMEM((nb, batch), jnp.int32),
            pltpu.VMEM((nb, batch), jnp.int32),
            pltpu.VMEM((nb, batch), jnp.float32),
            pltpu.VMEM((NBUF, batch, dh), jnp.float32),
            pltpu.VMEM((NBUF, batch, LANES), jnp.float32),
            pltpu.VMEM((stage, dh), jnp.float32),
            pltpu.VMEM((stage, LANES), jnp.float32),
            pltpu.VMEM_SHARED((n_pad, dh), jnp.float32),
            pltpu.VMEM_SHARED((n_pad, LANES), jnp.float32),
            [pltpu.SemaphoreType.DMA] * NBUF,
            [pltpu.SemaphoreType.DMA] * NBUF,
        ],
    )
    def sc_kernel(x_hbm, eb_hbm, ea_hbm, w_hbm, g_out, s_out,
                  idx_b, idx_a, w_v, rows_v, srow_v, stage_g, stage_s,
                  g_acc, s_acc, gsem, ssem):
        cid = lax.axis_index("c")
        sid = lax.axis_index("s")
        on_c0 = cid == 0
        zero16 = jnp.zeros((LANES,), jnp.float32)

        def zg(i, _):
            for j in range(dh // LANES):
                stage_g[i, pl.ds(j * LANES, LANES)] = zero16
            stage_s[i, :] = zero16
            return 0
        lax.fori_loop(0, stage, zg, 0)
        for t in range(n_stage):
            row0 = sid * rows_per_tile + t * stage
            pltpu.sync_copy(stage_g, g_acc.at[pl.ds(row0, stage)])
            pltpu.sync_copy(stage_s, s_acc.at[pl.ds(row0, stage)])
        plsc.subcore_barrier()

        pltpu.sync_copy(eb_hbm.at[sid], idx_b)
        pltpu.sync_copy(ea_hbm.at[sid], idx_a)
        pltpu.sync_copy(w_hbm.at[sid], w_v)

        def to_half_rows(k, _):
            for j in range(batch // LANES):
                sl = pl.ds(j * LANES, LANES)
                v = idx_b[k, sl]
                idx_b[k, sl] = v + v + cid
            return 0
        lax.fori_loop(0, nb, to_half_rows, 0)

        def issue_gather(k, p):
            pltpu.async_copy(x_hbm.at[idx_b.at[k]], rows_v.at[p], gsem[p])

        def wait_gather(k, p):
            pltpu.make_async_copy(x_hbm.at[idx_b.at[k]],
                                  rows_v.at[p], gsem[p]).wait()

        def issue_scatter(k, p):
            pltpu.async_copy(rows_v.at[p], g_acc.at[idx_a.at[k]],
                             ssem[p], add=True)

            @pl.when(on_c0)
            def _():
                pltpu.async_copy(srow_v.at[p], s_acc.at[idx_a.at[k]],
                                 ssem[p], add=True)

        def wait_scatter(k, p):
            pltpu.make_async_copy(rows_v.at[p], g_acc.at[idx_a.at[k]],
                                  ssem[p]).wait()

            @pl.when(on_c0)
            def _():
                pltpu.make_async_copy(srow_v.at[p], s_acc.at[idx_a.at[k]],
                                      ssem[p]).wait()

        iota16 = lax.iota(jnp.int32, LANES)

        def scale(k, p):
            for (off, lo, cnt) in groups:
                wvec = w_v[k, pl.ds(off, LANES)]
                wvs = [jnp.full((LANES,), wvec[lo + t], jnp.float32)
                       for t in range(cnt)]
                for t in range(cnt):
                    srow_v[p, off + lo + t, :] = wvs[t]
                for t in range(cnt):
                    for j in range(dh // LANES):
                        sl = pl.ds(j * LANES, LANES)
                        i = off + lo + t
                        rows_v[p, i, sl] = rows_v[p, i, sl] * wvs[t]

        issue_gather(0, 0)
        issue_gather(1, 1)
        n_it = nb // NBUF

        def body(m, _):
            for p in range(NBUF):
                k = m * NBUF + p
                q = (p + 2) % NBUF

                if p >= NBUF - 2:
                    wait_scatter(k - (NBUF - 2), q)
                else:
                    @pl.when(m >= 1)
                    def _():
                        wait_scatter(k - (NBUF - 2), q)

                if p < NBUF - 2:
                    issue_gather(k + 2, q)
                else:
                    @pl.when(m < n_it - 1)
                    def _():
                        issue_gather(k + 2, q)

                wait_gather(k, p)
                scale(k, p)
                issue_scatter(k, p)
            return 0
        lax.fori_loop(0, n_it, body, 0)
        for k in range(nb - (NBUF - 2), nb):
            wait_scatter(k, k % NBUF)
        plsc.subcore_barrier()

        for t in range(n_stage):
            row0 = sid * rows_per_tile + t * stage
            pltpu.sync_copy(g_acc.at[pl.ds(row0, stage)], stage_g)
            pltpu.sync_copy(stage_g, g_out.at[pl.ds(row0, stage), cid])

            @pl.when(on_c0)
            def _():
                pltpu.sync_copy(s_acc.at[pl.ds(row0, stage)], stage_s)
                pltpu.sync_copy(stage_s, s_out.at[pl.ds(row0, stage)])

    return sc_kernel


def _tc_body(x_ref, g_ref, s_ref, w_ref, b_ref, o_ref, *, in_c):
    w1 = w_ref[0:in_c, :]
    w2 = w_ref[in_c:2 * in_c, :]
    x = x_ref[...]
    g = g_ref[...]
    sv = s_ref[:, 0:1]
    h = jnp.dot(x, w1 - w2, preferred_element_type=jnp.float32) + b_ref[...]
    y = sv * h + jnp.dot(g, w2, preferred_element_type=jnp.float32)
    o_ref[...] = jnp.where(y >= 0, y, 0.3 * y)


def kernel(input_features, edges_a, edges_b, adj_weights, W, b):
    n, in_c = input_features.shape
    e = edges_a.shape[0]
    out_c = W.shape[1]
    dh = in_c // NC
    batch = 32
    nb = e // NS // batch

    x_view = input_features.reshape(n * NC, dh)
    eb_r = edges_b.reshape(NS, nb, batch)
    ea_r = edges_a.reshape(NS, nb, batch)
    w_r = adj_weights.reshape(NS, nb, batch)

    n_pad = ((n + NS * 128 - 1) // (NS * 128)) * (NS * 128)
    g_parts, s_part = _sc_segment_kernel(n_pad, e, dh, batch)(
        x_view, eb_r, ea_r, w_r)
    g_full = g_parts.reshape(n_pad, in_c)

    bn = 1000
    out = pl.pallas_call(
        functools.partial(_tc_body, in_c=in_c),
        grid=(n // bn,),
        in_specs=[
            pl.BlockSpec((bn, in_c), lambda i: (i, 0)),
            pl.BlockSpec((bn, in_c), lambda i: (i, 0)),
            pl.BlockSpec((bn, LANES), lambda i: (i, 0)),
            pl.BlockSpec((2 * in_c, out_c), lambda i: (0, 0)),
            pl.BlockSpec((1, out_c), lambda i: (0, 0)),
        ],
        out_specs=pl.BlockSpec((bn, out_c), lambda i: (i, 0)),
        out_shape=jax.ShapeDtypeStruct((n, out_c), jnp.float32),
    )(input_features, g_full, s_part, W, b.reshape(1, out_c))
    return out


# speedup vs baseline: 1.1294x; 1.0278x over previous
import functools

import jax
import jax.numpy as jnp
from jax import lax
from jax.experimental import pallas as pl
from jax.experimental.pallas import tpu as pltpu
from jax.experimental.pallas import tpu_sc as plsc

NC = 2
NS = 16
LANES = 16
NBUF = 5


def _sc_segment_kernel(n_pad: int, e: int, dh: int, batch: int):
    chunk = e // NS
    nb = chunk // batch
    rows_per_tile = n_pad // NS
    stage = 64
    n_stage = rows_per_tile // stage
    groups = [(g * LANES, 0, LANES) for g in range(batch // LANES)]
    if batch % LANES:
        groups.append((batch - LANES, LANES - batch % LANES, batch % LANES))

    mesh = plsc.VectorSubcoreMesh(
        core_axis_name="c", subcore_axis_name="s",
        num_cores=NC, num_subcores=NS)

    @functools.partial(
        pl.kernel,
        out_type=(
            jax.ShapeDtypeStruct((n_pad, NC, dh), jnp.float32),
            jax.ShapeDtypeStruct((n_pad, LANES), jnp.float32),
        ),
        mesh=mesh,
        compiler_params=pltpu.CompilerParams(
            use_tc_tiling_on_sc=False, needs_layout_passes=False),
        scratch_types=[
            pltpu.VMEM((nb, batch), jnp.int32),
            pltpu.VMEM((nb, batch), jnp.int32),
            pltpu.VMEM((nb, batch), jnp.float32),
            pltpu.VMEM((NBUF, batch, dh), jnp.float32),
            pltpu.VMEM((NBUF, batch, LANES), jnp.float32),
            pltpu.VMEM((stage, dh), jnp.float32),
            pltpu.VMEM((stage, LANES), jnp.float32),
            pltpu.VMEM_SHARED((n_pad, dh), jnp.float32),
            pltpu.VMEM_SHARED((n_pad, LANES), jnp.float32),
            [pltpu.SemaphoreType.DMA] * NBUF,
            [pltpu.SemaphoreType.DMA] * NBUF,
        ],
    )
    def sc_kernel(x_hbm, eb_hbm, ea_hbm, w_hbm, g_out, s_out,
                  idx_b, idx_a, w_v, rows_v, srow_v, stage_g, stage_s,
                  g_acc, s_acc, gsem, ssem):
        cid = lax.axis_index("c")
        sid = lax.axis_index("s")
        on_c0 = cid == 0
        zero16 = jnp.zeros((LANES,), jnp.float32)

        eb_d = pltpu.async_copy(eb_hbm.at[sid], idx_b, gsem[2])
        ea_d = pltpu.async_copy(ea_hbm.at[sid], idx_a, gsem[3])
        w_d = pltpu.async_copy(w_hbm.at[sid], w_v, gsem[4])

        def zg(i, _):
            for j in range(dh // LANES):
                stage_g[i, pl.ds(j * LANES, LANES)] = zero16
            stage_s[i, :] = zero16
            return 0
        lax.fori_loop(0, stage, zg, 0)
        for t in range(n_stage):
            row0 = sid * rows_per_tile + t * stage
            pltpu.sync_copy(stage_g, g_acc.at[pl.ds(row0, stage)])
            pltpu.sync_copy(stage_s, s_acc.at[pl.ds(row0, stage)])
        eb_d.wait()
        ea_d.wait()
        w_d.wait()
        plsc.subcore_barrier()

        def to_half_rows(k, _):
            for j in range(batch // LANES):
                sl = pl.ds(j * LANES, LANES)
                v = idx_b[k, sl]
                idx_b[k, sl] = v + v + cid
            return 0
        lax.fori_loop(0, nb, to_half_rows, 0)

        def issue_gather(k, p):
            pltpu.async_copy(x_hbm.at[idx_b.at[k]], rows_v.at[p], gsem[p])

        def wait_gather(k, p):
            pltpu.make_async_copy(x_hbm.at[idx_b.at[k]],
                                  rows_v.at[p], gsem[p]).wait()

        def issue_scatter(k, p):
            pltpu.async_copy(rows_v.at[p], g_acc.at[idx_a.at[k]],
                             ssem[p], add=True)

            @pl.when(on_c0)
            def _():
                pltpu.async_copy(srow_v.at[p], s_acc.at[idx_a.at[k]],
                                 ssem[p], add=True)

        def wait_scatter(k, p):
            pltpu.make_async_copy(rows_v.at[p], g_acc.at[idx_a.at[k]],
                                  ssem[p]).wait()

            @pl.when(on_c0)
            def _():
                pltpu.make_async_copy(srow_v.at[p], s_acc.at[idx_a.at[k]],
                                      ssem[p]).wait()

        iota16 = lax.iota(jnp.int32, LANES)

        def scale(k, p):
            for (off, lo, cnt) in groups:
                wvec = w_v[k, pl.ds(off, LANES)]
                wvs = [jnp.full((LANES,), wvec[lo + t], jnp.float32)
                       for t in range(cnt)]
                for t in range(cnt):
                    srow_v[p, off + lo + t, :] = wvs[t]
                for t in range(cnt):
                    for j in range(dh // LANES):
                        sl = pl.ds(j * LANES, LANES)
                        i = off + lo + t
                        rows_v[p, i, sl] = rows_v[p, i, sl] * wvs[t]

        issue_gather(0, 0)
        issue_gather(1, 1)
        n_it = nb // NBUF

        def body(m, _):
            for p in range(NBUF):
                k = m * NBUF + p
                q = (p + 2) % NBUF

                if p >= NBUF - 2:
                    wait_scatter(k - (NBUF - 2), q)
                else:
                    @pl.when(m >= 1)
                    def _():
                        wait_scatter(k - (NBUF - 2), q)

                if p < NBUF - 2:
                    issue_gather(k + 2, q)
                else:
                    @pl.when(m < n_it - 1)
                    def _():
                        issue_gather(k + 2, q)

                wait_gather(k, p)
                scale(k, p)
                issue_scatter(k, p)
            return 0
        lax.fori_loop(0, n_it, body, 0)
        for k in range(nb - (NBUF - 2), nb):
            wait_scatter(k, k % NBUF)
        plsc.subcore_barrier()

        for t in range(n_stage):
            row0 = sid * rows_per_tile + t * stage
            pltpu.sync_copy(g_acc.at[pl.ds(row0, stage)], stage_g)
            pltpu.sync_copy(stage_g, g_out.at[pl.ds(row0, stage), cid])

            @pl.when(on_c0)
            def _():
                pltpu.sync_copy(s_acc.at[pl.ds(row0, stage)], stage_s)
                pltpu.sync_copy(stage_s, s_out.at[pl.ds(row0, stage)])

    return sc_kernel


def _tc_body(x_ref, g_ref, s_ref, w_ref, b_ref, o_ref, *, in_c):
    w1 = w_ref[0:in_c, :]
    w2 = w_ref[in_c:2 * in_c, :]
    x = x_ref[...]
    g = g_ref[...]
    sv = s_ref[:, 0:1]
    h = jnp.dot(x, w1 - w2, preferred_element_type=jnp.float32) + b_ref[...]
    y = sv * h + jnp.dot(g, w2, preferred_element_type=jnp.float32)
    o_ref[...] = jnp.where(y >= 0, y, 0.3 * y)


def kernel(input_features, edges_a, edges_b, adj_weights, W, b):
    n, in_c = input_features.shape
    e = edges_a.shape[0]
    out_c = W.shape[1]
    dh = in_c // NC
    batch = 32
    nb = e // NS // batch

    x_view = input_features.reshape(n * NC, dh)
    eb_r = edges_b.reshape(NS, nb, batch)
    ea_r = edges_a.reshape(NS, nb, batch)
    w_r = adj_weights.reshape(NS, nb, batch)

    n_pad = ((n + NS * 128 - 1) // (NS * 128)) * (NS * 128)
    g_parts, s_part = _sc_segment_kernel(n_pad, e, dh, batch)(
        x_view, eb_r, ea_r, w_r)
    g_full = g_parts.reshape(n_pad, in_c)

    bn = 2000
    out = pl.pallas_call(
        functools.partial(_tc_body, in_c=in_c),
        grid=(n // bn,),
        in_specs=[
            pl.BlockSpec((bn, in_c), lambda i: (i, 0)),
            pl.BlockSpec((bn, in_c), lambda i: (i, 0)),
            pl.BlockSpec((bn, LANES), lambda i: (i, 0)),
            pl.BlockSpec((2 * in_c, out_c), lambda i: (0, 0)),
            pl.BlockSpec((1, out_c), lambda i: (0, 0)),
        ],
        out_specs=pl.BlockSpec((bn, out_c), lambda i: (i, 0)),
        out_shape=jax.ShapeDtypeStruct((n, out_c), jnp.float32),
    )(input_features, g_full, s_part, W, b.reshape(1, out_c))
    return out
